# baseline (speedup 1.0000x reference)
import jax
import jax.numpy as jnp
from jax import lax
from jax.experimental import pallas as pl
from jax.experimental.pallas import tpu as pltpu
from jax.experimental.pallas import tpu_sc as plsc

B = 8
S = 2048
D = 2048
DK = 256
NODE = 64
L = 64

NC = 2
NS = 16
NW = NC * NS
LANES = 16

GCHUNK = 128
SBLK = 512

_MESH = plsc.VectorSubcoreMesh(core_axis_name="c", subcore_axis_name="s",
                               num_cores=NC, num_subcores=NS)


def _proj_body(x_ref, w_ref, b_ref, p_ref):
    x = x_ref[0].astype(jnp.bfloat16)
    w = w_ref[...].astype(jnp.bfloat16)
    p = jnp.dot(x, w, preferred_element_type=jnp.float32)
    p_ref[0] = p + b_ref[...]


def _projection(seq_output, W_pre, b_pre2, nb):
    return pl.pallas_call(
        _proj_body,
        grid=(nb, S // SBLK),
        in_specs=[
            pl.BlockSpec((1, SBLK, D), lambda b, s: (b, s, 0)),
            pl.BlockSpec((D, DK), lambda b, s: (0, 0)),
            pl.BlockSpec((1, DK), lambda b, s: (0, 0)),
        ],
        out_specs=pl.BlockSpec((1, SBLK, DK), lambda b, s: (b, s, 0)),
        out_shape=jax.ShapeDtypeStruct((nb, S, DK), jnp.float32),
    )(seq_output, W_pre, b_pre2)


def _make_sc_gather(nb):
    rows = nb * NODE * L
    rows_per_w = rows // NW

    def body(p2, idxf, g_out, idx_v, buf, sem):
        wid = lax.axis_index("s") * NC + lax.axis_index("c")
        base = wid * rows_per_w
        pltpu.sync_copy(idxf.at[pl.ds(base, rows_per_w)], idx_v)
        off = (wid // (NODE * L // rows_per_w)) * S
        offv = jnp.full((LANES,), 0, jnp.int32) + off
        for j in range(rows_per_w // LANES):
            sl = pl.ds(j * LANES, LANES)
            idx_v[sl] = idx_v[sl] + offv
        for c in range(rows_per_w // GCHUNK):
            pltpu.async_copy(
                p2.at[idx_v.at[pl.ds(c * GCHUNK, GCHUNK)]], buf, sem).wait()
            pltpu.sync_copy(buf, g_out.at[pl.ds(base + c * GCHUNK, GCHUNK)])

    return pl.kernel(
        body,
        out_type=jax.ShapeDtypeStruct((rows, DK), jnp.float32),
        mesh=_MESH,
        scratch_types=[
            pltpu.VMEM((rows_per_w,), jnp.int32),
            pltpu.VMEM((GCHUNK, DK), jnp.float32),
            pltpu.SemaphoreType.DMA,
        ],
    )


def _attn_body(g_ref, h_ref, wq_ref, wv_ref, s_ref):
    g = g_ref[...]
    t = jnp.dot(h_ref[0], wq_ref[...],
                preferred_element_type=jnp.float32)
    wv = wv_ref[0]
    tmp = jnp.tanh(g + t[0][None, None, :])
    e = jnp.sum(tmp * wv[None, None, :], axis=2)
    m = jnp.max(e, axis=1, keepdims=True)
    p = jnp.exp(e - m)
    s_ref[...] = p / jnp.sum(p, axis=1, keepdims=True)


def _attention_scores(G3, hidden3, W_q, wv2, nb):
    return pl.pallas_call(
        _attn_body,
        grid=(nb,),
        in_specs=[
            pl.BlockSpec((NODE, L, DK), lambda b: (b, 0, 0)),
            pl.BlockSpec((1, 1, D), lambda b: (b, 0, 0)),
            pl.BlockSpec((D, DK), lambda b: (0, 0)),
            pl.BlockSpec((1, DK), lambda b: (0, 0)),
        ],
        out_specs=pl.BlockSpec((NODE, L), lambda b: (b, 0)),
        out_shape=jax.ShapeDtypeStruct((nb * NODE, L), jnp.float32),
    )(G3, hidden3, W_q, wv2)


def _make_sc_scatter(nb):
    n_bags = nb * NODE
    bags_per_w = n_bags // NW
    fold = LANES // bags_per_w
    n_vals = L * bags_per_w

    def body(score_t, idx_t, a_out, sc_v, ix_v, acc):
        wid = lax.axis_index("s") * NC + lax.axis_index("c")
        base = wid * bags_per_w
        pltpu.sync_copy(score_t.at[wid], sc_v)
        pltpu.sync_copy(idx_t.at[wid], ix_v)
        zeros = jnp.zeros((LANES,), jnp.float32)

        def zero_row(j, _):
            for i in range(LANES):
                acc[i, pl.ds(j * LANES, LANES)] = zeros
            return 0

        lax.fori_loop(0, S // LANES, zero_row, 0)
        rows16 = lax.iota(jnp.int32, LANES)
        for v in range(n_vals // LANES):
            iv = ix_v[pl.ds(v * LANES, LANES)]
            sv = sc_v[pl.ds(v * LANES, LANES)]
            plsc.addupdate_scatter(acc, [rows16, iv], sv)
        if fold > 1:
            def fold_row(j, _):
                sl = pl.ds(j * LANES, LANES)
                for i in range(bags_per_w):
                    s = acc[i, sl]
                    for f in range(1, fold):
                        s = s + acc[i + f * bags_per_w, sl]
                    acc[i, sl] = s
                return 0

            lax.fori_loop(0, S // LANES, fold_row, 0)
        pltpu.sync_copy(acc.at[pl.ds(0, bags_per_w)],
                        a_out.at[pl.ds(base, bags_per_w)])

    return pl.kernel(
        body,
        out_type=jax.ShapeDtypeStruct((n_bags, S), jnp.float32),
        mesh=_MESH,
        scratch_types=[
            pltpu.VMEM((n_vals,), jnp.float32),
            pltpu.VMEM((n_vals,), jnp.int32),
            pltpu.VMEM((LANES, S), jnp.float32),
        ],
        compiler_params=pltpu.CompilerParams(use_tc_tiling_on_sc=False,
                                             needs_layout_passes=False),
    )


def _ctx_body(a_ref, x_ref, nl_ref, n_ref, m_ref):
    a = a_ref[0]
    x = x_ref[0]
    n_ref[0] = jnp.dot(a.astype(jnp.bfloat16), x.astype(jnp.bfloat16),
                       preferred_element_type=jnp.float32)
    pos = lax.broadcasted_iota(jnp.int32, (1, 1, NODE), 2)
    m_ref[...] = (pos < nl_ref[0]).astype(jnp.float32)


def _context(A3, seq_output, node_lengths, nb):
    return pl.pallas_call(
        _ctx_body,
        grid=(nb,),
        in_specs=[
            pl.BlockSpec((1, NODE, S), lambda b: (b, 0, 0)),
            pl.BlockSpec((1, S, D), lambda b: (b, 0, 0)),
            pl.BlockSpec(memory_space=pltpu.SMEM),
        ],
        out_specs=[
            pl.BlockSpec((1, NODE, D), lambda b: (b, 0, 0)),
            pl.BlockSpec((1, 1, NODE), lambda b: (b, 0, 0)),
        ],
        out_shape=[
            jax.ShapeDtypeStruct((nb, NODE, D), jnp.float32),
            jax.ShapeDtypeStruct((nb, 1, NODE), jnp.float32),
        ],
    )(A3, seq_output, node_lengths)


def _chunk(seq_c, hidden2_c, index_c, node_lengths_c, W_pre, b_pre2, W_q,
           wv2, nb):
    rows = nb * NODE * L
    n_bags = nb * NODE
    bags_per_w = n_bags // NW
    P = _projection(seq_c, W_pre, b_pre2, nb)
    P2 = P.reshape(nb * S, DK)
    idx_flat = index_c.reshape(rows).astype(jnp.int32)
    G = _make_sc_gather(nb)(P2, idx_flat)
    G3 = G.reshape(n_bags, L, DK)
    score = _attention_scores(G3, hidden2_c.reshape(nb, 1, D), W_q, wv2, nb)
    idx2 = index_c.reshape(n_bags, L).astype(jnp.int32)
    score3 = (score.reshape(NW, bags_per_w, L).transpose(0, 2, 1)
              .reshape(NW, L * bags_per_w))
    idx3 = (idx2.reshape(NW, bags_per_w, L).transpose(0, 2, 1)
            .reshape(NW, L * bags_per_w))
    A = _make_sc_scatter(nb)(score3, idx3)
    A3 = A.reshape(nb, NODE, S)
    nodes, mask3 = _context(A3, seq_c, node_lengths_c, nb)
    return nodes, mask3.reshape(nb, NODE)


NCHUNK = 2


def kernel(seq_output, hidden, index, lengths, node_lengths, feat_seqs,
           node_type, W_pre, b_pre, W_q, W_v, max_length):
    hidden2 = jnp.transpose(hidden, (1, 0, 2)).reshape(B, D)
    b_pre2 = b_pre.reshape(1, DK)
    wv2 = W_v.reshape(1, DK)
    nb = B // NCHUNK
    nodes_parts, mask_parts = [], []
    for h in range(NCHUNK):
        sl = slice(h * nb, (h + 1) * nb)
        nodes_h, mask_h = _chunk(seq_output[sl], hidden2[sl], index[sl],
                                 node_lengths[sl], W_pre, b_pre2, W_q,
                                 wv2, nb)
        nodes_parts.append(nodes_h)
        mask_parts.append(mask_h)
    nodes = jnp.concatenate(nodes_parts, axis=0)
    nodes_mask = jnp.concatenate(mask_parts, axis=0)
    return nodes, nodes_mask, hidden2


# speedup vs baseline: 1.9087x; 1.9087x over previous
import jax
import jax.numpy as jnp
from jax import lax
from jax.experimental import pallas as pl
from jax.experimental.pallas import tpu as pltpu
from jax.experimental.pallas import tpu_sc as plsc

B = 8
S = 2048
D = 2048
DK = 256
NODE = 64
L = 64
N = B * NODE

NC = 2
NS = 16
NW = NC * NS
LANES = 16

BAGS_PER_W = N // NW
SBLK = 512

_MESH = plsc.VectorSubcoreMesh(core_axis_name="c", subcore_axis_name="s",
                               num_cores=NC, num_subcores=NS)


def _energy_body(x_ref, w_ref, b_ref, h_ref, wq_ref, wv_ref, e_ref):
    x = x_ref[0].astype(jnp.bfloat16)
    w = w_ref[...].astype(jnp.bfloat16)
    p = jnp.dot(x, w, preferred_element_type=jnp.float32)
    t = jnp.dot(h_ref[0].astype(jnp.bfloat16),
                wq_ref[...].astype(jnp.bfloat16),
                preferred_element_type=jnp.float32)
    tmp = jnp.tanh(p + b_ref[...] + t)
    e_ref[0] = jnp.sum(tmp * wv_ref[...], axis=1, keepdims=True)


def _energy(seq_output, W_pre, b_pre2, hidden3, W_q, wv2):
    return pl.pallas_call(
        _energy_body,
        grid=(B, S // SBLK),
        in_specs=[
            pl.BlockSpec((1, SBLK, D), lambda b, s: (b, s, 0)),
            pl.BlockSpec((D, DK), lambda b, s: (0, 0)),
            pl.BlockSpec((1, DK), lambda b, s: (0, 0)),
            pl.BlockSpec((1, 1, D), lambda b, s: (b, 0, 0)),
            pl.BlockSpec((D, DK), lambda b, s: (0, 0)),
            pl.BlockSpec((1, DK), lambda b, s: (0, 0)),
        ],
        out_specs=pl.BlockSpec((1, SBLK, 1), lambda b, s: (b, s, 0)),
        out_shape=jax.ShapeDtypeStruct((B, S, 1), jnp.float32),
    )(seq_output, W_pre, b_pre2, hidden3, W_q, wv2)


def _sc_attn_body(e_flat, idx_t, a_out, e_row, ix_v, sc_v, acc):
    wid = lax.axis_index("s") * NC + lax.axis_index("c")
    sample = wid // (NW // B)
    pltpu.sync_copy(e_flat.at[pl.ds(sample * S, S)], e_row)
    pltpu.sync_copy(idx_t.at[wid], ix_v)
    zeros = jnp.zeros((LANES,), jnp.float32)

    def zero_row(j, _):
        for i in range(LANES):
            acc[i, pl.ds(j * LANES, LANES)] = zeros
        return 0

    lax.fori_loop(0, S // LANES, zero_row, 0)

    for i in range(BAGS_PER_W):
        base = i * L
        ev = [plsc.load_gather(e_row, [ix_v[pl.ds(base + k * LANES, LANES)]])
              for k in range(L // LANES)]
        m = ev[0]
        for v in ev[1:]:
            m = jnp.maximum(m, v)
        mm = jnp.max(m)
        pv = [jnp.exp(v - mm) for v in ev]
        ssum = pv[0]
        for v in pv[1:]:
            ssum = ssum + v
        rv = (zeros + 1.0) / (zeros + jnp.sum(ssum))
        for k in range(L // LANES):
            sc_v[pl.ds(base + k * LANES, LANES)] = pv[k] * rv

    rows16 = lax.iota(jnp.int32, LANES)
    pos0 = rows16 * L
    for l in range(L):
        pos = pos0 + l
        iv = plsc.load_gather(ix_v, [pos])
        sv = plsc.load_gather(sc_v, [pos])
        plsc.addupdate_scatter(acc, [rows16, iv], sv)
    pltpu.sync_copy(acc, a_out.at[pl.ds(wid * BAGS_PER_W, BAGS_PER_W)])


def _sc_attention(e_flat, idx_t):
    k = pl.kernel(
        _sc_attn_body,
        out_type=jax.ShapeDtypeStruct((N, S), jnp.float32),
        mesh=_MESH,
        scratch_types=[
            pltpu.VMEM((S,), jnp.float32),
            pltpu.VMEM((BAGS_PER_W * L,), jnp.int32),
            pltpu.VMEM((BAGS_PER_W * L,), jnp.float32),
            pltpu.VMEM((LANES, S), jnp.float32),
        ],
        compiler_params=pltpu.CompilerParams(use_tc_tiling_on_sc=False,
                                             needs_layout_passes=False),
    )
    return k(e_flat, idx_t)


def _ctx_body(a_ref, x_ref, nl_ref, n_ref, m_ref):
    a = a_ref[0]
    x = x_ref[0]
    n_ref[0] = jnp.dot(a.astype(jnp.bfloat16), x.astype(jnp.bfloat16),
                       preferred_element_type=jnp.float32)
    pos = lax.broadcasted_iota(jnp.int32, (1, 1, NODE), 2)
    m_ref[...] = (pos < nl_ref[0]).astype(jnp.float32)


def _context(A3, seq_output, node_lengths):
    return pl.pallas_call(
        _ctx_body,
        grid=(B,),
        in_specs=[
            pl.BlockSpec((1, NODE, S), lambda b: (b, 0, 0)),
            pl.BlockSpec((1, S, D), lambda b: (b, 0, 0)),
            pl.BlockSpec(memory_space=pltpu.SMEM),
        ],
        out_specs=[
            pl.BlockSpec((1, NODE, D), lambda b: (b, 0, 0)),
            pl.BlockSpec((1, 1, NODE), lambda b: (b, 0, 0)),
        ],
        out_shape=[
            jax.ShapeDtypeStruct((B, NODE, D), jnp.float32),
            jax.ShapeDtypeStruct((B, 1, NODE), jnp.float32),
        ],
    )(A3, seq_output, node_lengths)


def kernel(seq_output, hidden, index, lengths, node_lengths, feat_seqs,
           node_type, W_pre, b_pre, W_q, W_v, max_length):
    hidden2 = jnp.transpose(hidden, (1, 0, 2)).reshape(B, D)
    e3 = _energy(seq_output, W_pre, b_pre.reshape(1, DK),
                 hidden2.reshape(B, 1, D), W_q, W_v.reshape(1, DK))
    e_flat = e3.reshape(B * S)
    idx_t = index.reshape(NW, BAGS_PER_W * L).astype(jnp.int32)
    A = _sc_attention(e_flat, idx_t)
    A3 = A.reshape(B, NODE, S)
    nodes, mask3 = _context(A3, seq_output, node_lengths)
    return nodes, mask3.reshape(B, NODE), hidden2


# speedup vs baseline: 2.0451x; 1.0715x over previous
import jax
import jax.numpy as jnp
from jax import lax
from jax.experimental import pallas as pl
from jax.experimental.pallas import tpu as pltpu
from jax.experimental.pallas import tpu_sc as plsc

B = 8
S = 2048
D = 2048
DK = 256
NODE = 64
L = 64
N = B * NODE

NC = 2
NS = 16
NW = NC * NS
LANES = 16

BAGS_PER_W = N // NW
SBLK = 1024

_MESH = plsc.VectorSubcoreMesh(core_axis_name="c", subcore_axis_name="s",
                               num_cores=NC, num_subcores=NS)


def _energy_body(x_ref, w_ref, b_ref, h_ref, wq_ref, wv_ref, e_ref):
    x = x_ref[0].astype(jnp.bfloat16)
    w = w_ref[...].astype(jnp.bfloat16)
    p = jnp.dot(x, w, preferred_element_type=jnp.float32)
    t = jnp.dot(h_ref[0].astype(jnp.bfloat16),
                wq_ref[...].astype(jnp.bfloat16),
                preferred_element_type=jnp.float32)
    tmp = jnp.tanh(p + b_ref[...] + t)
    e_ref[0] = jnp.sum(tmp * wv_ref[...], axis=1, keepdims=True)


def _energy(seq_output, W_pre, b_pre2, hidden3, W_q, wv2):
    return pl.pallas_call(
        _energy_body,
        grid=(B, S // SBLK),
        in_specs=[
            pl.BlockSpec((1, SBLK, D), lambda b, s: (b, s, 0)),
            pl.BlockSpec((D, DK), lambda b, s: (0, 0)),
            pl.BlockSpec((1, DK), lambda b, s: (0, 0)),
            pl.BlockSpec((1, 1, D), lambda b, s: (b, 0, 0)),
            pl.BlockSpec((D, DK), lambda b, s: (0, 0)),
            pl.BlockSpec((1, DK), lambda b, s: (0, 0)),
        ],
        out_specs=pl.BlockSpec((1, SBLK, 1), lambda b, s: (b, s, 0)),
        out_shape=jax.ShapeDtypeStruct((B, S, 1), jnp.float32),
    )(seq_output, W_pre, b_pre2, hidden3, W_q, wv2)


def _sc_attn_body(e_flat, idx_t, a_out, e_row, ix_v, sc_v, acc):
    wid = lax.axis_index("s") * NC + lax.axis_index("c")
    sample = wid // (NW // B)
    pltpu.sync_copy(e_flat.at[pl.ds(sample * S, S)], e_row)
    pltpu.sync_copy(idx_t.at[wid], ix_v)
    zeros = jnp.zeros((LANES,), jnp.float32)

    def zero_row(j, _):
        for i in range(LANES):
            acc[i, pl.ds(j * LANES, LANES)] = zeros
        return 0

    lax.fori_loop(0, S // LANES, zero_row, 0)

    for i in range(BAGS_PER_W):
        base = i * L
        ev = [plsc.load_gather(e_row, [ix_v[pl.ds(base + k * LANES, LANES)]])
              for k in range(L // LANES)]
        m = ev[0]
        for v in ev[1:]:
            m = jnp.maximum(m, v)
        mm = jnp.max(m)
        pv = [jnp.exp(v - mm) for v in ev]
        ssum = pv[0]
        for v in pv[1:]:
            ssum = ssum + v
        rv = (zeros + 1.0) / (zeros + jnp.sum(ssum))
        for k in range(L // LANES):
            sc_v[pl.ds(base + k * LANES, LANES)] = pv[k] * rv

    rows16 = lax.iota(jnp.int32, LANES)
    pos0 = rows16 * L
    for l in range(L):
        pos = pos0 + l
        iv = plsc.load_gather(ix_v, [pos])
        sv = plsc.load_gather(sc_v, [pos])
        plsc.addupdate_scatter(acc, [rows16, iv], sv)
    pltpu.sync_copy(acc, a_out.at[pl.ds(wid * BAGS_PER_W, BAGS_PER_W)])


def _sc_attention(e_flat, idx_t):
    k = pl.kernel(
        _sc_attn_body,
        out_type=jax.ShapeDtypeStruct((N, S), jnp.float32),
        mesh=_MESH,
        scratch_types=[
            pltpu.VMEM((S,), jnp.float32),
            pltpu.VMEM((BAGS_PER_W * L,), jnp.int32),
            pltpu.VMEM((BAGS_PER_W * L,), jnp.float32),
            pltpu.VMEM((LANES, S), jnp.float32),
        ],
        compiler_params=pltpu.CompilerParams(use_tc_tiling_on_sc=False,
                                             needs_layout_passes=False),
    )
    return k(e_flat, idx_t)


def _ctx_body(a_ref, x_ref, nl_ref, n_ref, m_ref):
    a = a_ref[0]
    x = x_ref[0]
    n_ref[0] = jnp.dot(a.astype(jnp.bfloat16), x.astype(jnp.bfloat16),
                       preferred_element_type=jnp.float32)
    pos = lax.broadcasted_iota(jnp.int32, (1, 1, NODE), 2)
    m_ref[...] = (pos < nl_ref[0]).astype(jnp.float32)


def _context(A3, seq_output, node_lengths):
    return pl.pallas_call(
        _ctx_body,
        grid=(B,),
        in_specs=[
            pl.BlockSpec((1, NODE, S), lambda b: (b, 0, 0)),
            pl.BlockSpec((1, S, D), lambda b: (b, 0, 0)),
            pl.BlockSpec(memory_space=pltpu.SMEM),
        ],
        out_specs=[
            pl.BlockSpec((1, NODE, D), lambda b: (b, 0, 0)),
            pl.BlockSpec((1, 1, NODE), lambda b: (b, 0, 0)),
        ],
        out_shape=[
            jax.ShapeDtypeStruct((B, NODE, D), jnp.float32),
            jax.ShapeDtypeStruct((B, 1, NODE), jnp.float32),
        ],
    )(A3, seq_output, node_lengths)


def kernel(seq_output, hidden, index, lengths, node_lengths, feat_seqs,
           node_type, W_pre, b_pre, W_q, W_v, max_length):
    hidden2 = jnp.transpose(hidden, (1, 0, 2)).reshape(B, D)
    e3 = _energy(seq_output, W_pre, b_pre.reshape(1, DK),
                 hidden2.reshape(B, 1, D), W_q, W_v.reshape(1, DK))
    e_flat = e3.reshape(B * S)
    idx_t = index.reshape(NW, BAGS_PER_W * L).astype(jnp.int32)
    A = _sc_attention(e_flat, idx_t)
    A3 = A.reshape(B, NODE, S)
    nodes, mask3 = _context(A3, seq_output, node_lengths)
    return nodes, mask3.reshape(B, NODE), hidden2


# speedup vs baseline: 2.0531x; 1.0039x over previous
import jax
import jax.numpy as jnp
from jax import lax
from jax.experimental import pallas as pl
from jax.experimental.pallas import tpu as pltpu
from jax.experimental.pallas import tpu_sc as plsc

B = 8
S = 2048
D = 2048
DK = 256
NODE = 64
L = 64
N = B * NODE

NC = 2
NS = 16
NW = NC * NS
LANES = 16

BAGS_PER_W = N // NW
SBLK = 2048

_MESH = plsc.VectorSubcoreMesh(core_axis_name="c", subcore_axis_name="s",
                               num_cores=NC, num_subcores=NS)


def _energy_body(x_ref, w_ref, b_ref, h_ref, wq_ref, wv_ref, e_ref):
    x = x_ref[0].astype(jnp.bfloat16)
    w = w_ref[...].astype(jnp.bfloat16)
    p = jnp.dot(x, w, preferred_element_type=jnp.float32)
    t = jnp.dot(h_ref[0].astype(jnp.bfloat16),
                wq_ref[...].astype(jnp.bfloat16),
                preferred_element_type=jnp.float32)
    tmp = jnp.tanh(p + b_ref[...] + t)
    e_ref[0] = jnp.sum(tmp * wv_ref[...], axis=1, keepdims=True)


def _energy(seq_output, W_pre, b_pre2, hidden3, W_q, wv2):
    return pl.pallas_call(
        _energy_body,
        grid=(B, S // SBLK),
        in_specs=[
            pl.BlockSpec((1, SBLK, D), lambda b, s: (b, s, 0)),
            pl.BlockSpec((D, DK), lambda b, s: (0, 0)),
            pl.BlockSpec((1, DK), lambda b, s: (0, 0)),
            pl.BlockSpec((1, 1, D), lambda b, s: (b, 0, 0)),
            pl.BlockSpec((D, DK), lambda b, s: (0, 0)),
            pl.BlockSpec((1, DK), lambda b, s: (0, 0)),
        ],
        out_specs=pl.BlockSpec((1, SBLK, 1), lambda b, s: (b, s, 0)),
        out_shape=jax.ShapeDtypeStruct((B, S, 1), jnp.float32),
    )(seq_output, W_pre, b_pre2, hidden3, W_q, wv2)


def _sc_attn_body(e_flat, idx_t, a_out, e_row, ix_v, sc_v, acc):
    wid = lax.axis_index("s") * NC + lax.axis_index("c")
    sample = wid // (NW // B)
    pltpu.sync_copy(e_flat.at[pl.ds(sample * S, S)], e_row)
    pltpu.sync_copy(idx_t.at[wid], ix_v)
    zeros = jnp.zeros((LANES,), jnp.float32)

    def zero_row(j, _):
        for i in range(LANES):
            acc[i, pl.ds(j * LANES, LANES)] = zeros
        return 0

    lax.fori_loop(0, S // LANES, zero_row, 0)

    for i in range(BAGS_PER_W):
        base = i * L
        ev = [plsc.load_gather(e_row, [ix_v[pl.ds(base + k * LANES, LANES)]])
              for k in range(L // LANES)]
        m = ev[0]
        for v in ev[1:]:
            m = jnp.maximum(m, v)
        mm = jnp.max(m)
        pv = [jnp.exp(v - mm) for v in ev]
        ssum = pv[0]
        for v in pv[1:]:
            ssum = ssum + v
        rv = (zeros + 1.0) / (zeros + jnp.sum(ssum))
        for k in range(L // LANES):
            sc_v[pl.ds(base + k * LANES, LANES)] = pv[k] * rv

    rows16 = lax.iota(jnp.int32, LANES)
    pos0 = rows16 * L
    for l in range(L):
        pos = pos0 + l
        iv = plsc.load_gather(ix_v, [pos])
        sv = plsc.load_gather(sc_v, [pos])
        plsc.addupdate_scatter(acc, [rows16, iv], sv)
    pltpu.sync_copy(acc, a_out.at[pl.ds(wid * BAGS_PER_W, BAGS_PER_W)])


def _sc_attention(e_flat, idx_t):
    k = pl.kernel(
        _sc_attn_body,
        out_type=jax.ShapeDtypeStruct((N, S), jnp.float32),
        mesh=_MESH,
        scratch_types=[
            pltpu.VMEM((S,), jnp.float32),
            pltpu.VMEM((BAGS_PER_W * L,), jnp.int32),
            pltpu.VMEM((BAGS_PER_W * L,), jnp.float32),
            pltpu.VMEM((LANES, S), jnp.float32),
        ],
        compiler_params=pltpu.CompilerParams(use_tc_tiling_on_sc=False,
                                             needs_layout_passes=False),
    )
    return k(e_flat, idx_t)


def _ctx_body(a_ref, x_ref, nl_ref, n_ref, m_ref):
    a = a_ref[0]
    x = x_ref[0]
    n_ref[0] = jnp.dot(a.astype(jnp.bfloat16), x.astype(jnp.bfloat16),
                       preferred_element_type=jnp.float32)
    pos = lax.broadcasted_iota(jnp.int32, (1, 1, NODE), 2)
    m_ref[...] = (pos < nl_ref[0]).astype(jnp.float32)


def _context(A3, seq_output, node_lengths):
    return pl.pallas_call(
        _ctx_body,
        grid=(B,),
        in_specs=[
            pl.BlockSpec((1, NODE, S), lambda b: (b, 0, 0)),
            pl.BlockSpec((1, S, D), lambda b: (b, 0, 0)),
            pl.BlockSpec(memory_space=pltpu.SMEM),
        ],
        out_specs=[
            pl.BlockSpec((1, NODE, D), lambda b: (b, 0, 0)),
            pl.BlockSpec((1, 1, NODE), lambda b: (b, 0, 0)),
        ],
        out_shape=[
            jax.ShapeDtypeStruct((B, NODE, D), jnp.float32),
            jax.ShapeDtypeStruct((B, 1, NODE), jnp.float32),
        ],
    )(A3, seq_output, node_lengths)


def kernel(seq_output, hidden, index, lengths, node_lengths, feat_seqs,
           node_type, W_pre, b_pre, W_q, W_v, max_length):
    hidden2 = jnp.transpose(hidden, (1, 0, 2)).reshape(B, D)
    e3 = _energy(seq_output, W_pre, b_pre.reshape(1, DK),
                 hidden2.reshape(B, 1, D), W_q, W_v.reshape(1, DK))
    e_flat = e3.reshape(B * S)
    idx_t = index.reshape(NW, BAGS_PER_W * L).astype(jnp.int32)
    A = _sc_attention(e_flat, idx_t)
    A3 = A.reshape(B, NODE, S)
    nodes, mask3 = _context(A3, seq_output, node_lengths)
    return nodes, mask3.reshape(B, NODE), hidden2


# speedup vs baseline: 2.0551x; 1.0009x over previous
import jax
import jax.numpy as jnp
from jax import lax
from jax.experimental import pallas as pl
from jax.experimental.pallas import tpu as pltpu
from jax.experimental.pallas import tpu_sc as plsc

B = 8
S = 2048
D = 2048
DK = 256
NODE = 64
L = 64
N = B * NODE

NC = 2
NS = 16
NW = NC * NS
LANES = 16

BAGS_PER_W = N // NW
SBLK = 1024

_MESH = plsc.VectorSubcoreMesh(core_axis_name="c", subcore_axis_name="s",
                               num_cores=NC, num_subcores=NS)


def _energy_body(x_ref, w_ref, b_ref, h_ref, wq_ref, wv_ref, e_ref):
    x = x_ref[0].astype(jnp.bfloat16)
    w = w_ref[...].astype(jnp.bfloat16)
    p = jnp.dot(x, w, preferred_element_type=jnp.float32)
    t = jnp.dot(h_ref[0].astype(jnp.bfloat16),
                wq_ref[...].astype(jnp.bfloat16),
                preferred_element_type=jnp.float32)
    tmp = jnp.tanh(p + b_ref[...] + t)
    e_ref[0] = jnp.sum(tmp * wv_ref[...], axis=1, keepdims=True)


def _energy(seq_output, W_pre, b_pre2, hidden3, W_q, wv2):
    return pl.pallas_call(
        _energy_body,
        grid=(B, S // SBLK),
        in_specs=[
            pl.BlockSpec((1, SBLK, D), lambda b, s: (b, s, 0)),
            pl.BlockSpec((D, DK), lambda b, s: (0, 0)),
            pl.BlockSpec((1, DK), lambda b, s: (0, 0)),
            pl.BlockSpec((1, 1, D), lambda b, s: (b, 0, 0)),
            pl.BlockSpec((D, DK), lambda b, s: (0, 0)),
            pl.BlockSpec((1, DK), lambda b, s: (0, 0)),
        ],
        out_specs=pl.BlockSpec((1, SBLK, 1), lambda b, s: (b, s, 0)),
        out_shape=jax.ShapeDtypeStruct((B, S, 1), jnp.float32),
        compiler_params=pltpu.CompilerParams(
            vmem_limit_bytes=100 * 1024 * 1024),
    )(seq_output, W_pre, b_pre2, hidden3, W_q, wv2)


def _sc_attn_body(e_flat, idx_t, a_out, e_row, ix_v, sc_v, acc):
    wid = lax.axis_index("s") * NC + lax.axis_index("c")
    sample = wid // (NW // B)
    pltpu.sync_copy(e_flat.at[pl.ds(sample * S, S)], e_row)
    pltpu.sync_copy(idx_t.at[wid], ix_v)
    zeros = jnp.zeros((LANES,), jnp.float32)

    def zero_row(j, _):
        for i in range(LANES):
            acc[i, pl.ds(j * LANES, LANES)] = zeros
        return 0

    lax.fori_loop(0, S // LANES, zero_row, 0)

    def bag_softmax(i, _):
        base = i * L
        ev = [plsc.load_gather(e_row, [ix_v[pl.ds(base + k * LANES, LANES)]])
              for k in range(L // LANES)]
        m = ev[0]
        for v in ev[1:]:
            m = jnp.maximum(m, v)
        mm = jnp.max(m)
        pv = [jnp.exp(v - mm) for v in ev]
        ssum = pv[0]
        for v in pv[1:]:
            ssum = ssum + v
        rv = (zeros + 1.0) / (zeros + jnp.sum(ssum))
        for k in range(L // LANES):
            sc_v[pl.ds(base + k * LANES, LANES)] = pv[k] * rv
        return 0

    lax.fori_loop(0, BAGS_PER_W, bag_softmax, 0)

    rows16 = lax.iota(jnp.int32, LANES)
    pos0 = rows16 * L

    def scatter_l(l, _):
        pos = pos0 + l
        iv = plsc.load_gather(ix_v, [pos])
        sv = plsc.load_gather(sc_v, [pos])
        plsc.addupdate_scatter(acc, [rows16, iv], sv)
        return 0

    lax.fori_loop(0, L, scatter_l, 0)
    pltpu.sync_copy(acc, a_out.at[pl.ds(wid * BAGS_PER_W, BAGS_PER_W)])


def _sc_attention(e_flat, idx_t):
    k = pl.kernel(
        _sc_attn_body,
        out_type=jax.ShapeDtypeStruct((N, S), jnp.float32),
        mesh=_MESH,
        scratch_types=[
            pltpu.VMEM((S,), jnp.float32),
            pltpu.VMEM((BAGS_PER_W * L,), jnp.int32),
            pltpu.VMEM((BAGS_PER_W * L,), jnp.float32),
            pltpu.VMEM((LANES, S), jnp.float32),
        ],
        compiler_params=pltpu.CompilerParams(use_tc_tiling_on_sc=False,
                                             needs_layout_passes=False,
                                             disable_bounds_checks=True),
    )
    return k(e_flat, idx_t)


def _ctx_body(a_ref, x_ref, nl_ref, n_ref, m_ref):
    a = a_ref[0]
    x = x_ref[0]
    n_ref[0] = jnp.dot(a.astype(jnp.bfloat16), x.astype(jnp.bfloat16),
                       preferred_element_type=jnp.float32)
    pos = lax.broadcasted_iota(jnp.int32, (1, 1, NODE), 2)
    m_ref[...] = (pos < nl_ref[0]).astype(jnp.float32)


def _context(A3, seq_output, node_lengths):
    return pl.pallas_call(
        _ctx_body,
        grid=(B,),
        in_specs=[
            pl.BlockSpec((1, NODE, S), lambda b: (b, 0, 0)),
            pl.BlockSpec((1, S, D), lambda b: (b, 0, 0)),
            pl.BlockSpec(memory_space=pltpu.SMEM),
        ],
        out_specs=[
            pl.BlockSpec((1, NODE, D), lambda b: (b, 0, 0)),
            pl.BlockSpec((1, 1, NODE), lambda b: (b, 0, 0)),
        ],
        out_shape=[
            jax.ShapeDtypeStruct((B, NODE, D), jnp.float32),
            jax.ShapeDtypeStruct((B, 1, NODE), jnp.float32),
        ],
    )(A3, seq_output, node_lengths)


def kernel(seq_output, hidden, index, lengths, node_lengths, feat_seqs,
           node_type, W_pre, b_pre, W_q, W_v, max_length):
    hidden2 = jnp.transpose(hidden, (1, 0, 2)).reshape(B, D)
    e3 = _energy(seq_output, W_pre, b_pre.reshape(1, DK),
                 hidden2.reshape(B, 1, D), W_q, W_v.reshape(1, DK))
    e_flat = e3.reshape(B * S)
    idx_t = index.reshape(NW, BAGS_PER_W * L).astype(jnp.int32)
    A = _sc_attention(e_flat, idx_t)
    A3 = A.reshape(B, NODE, S)
    nodes, mask3 = _context(A3, seq_output, node_lengths)
    return nodes, mask3.reshape(B, NODE), hidden2
